# baseline (device time: 80999 ns/iter reference)
import contextlib
import os

import jax
import jax.numpy as jnp
from jax import lax
from jax.experimental import pallas as pl
from jax.experimental.pallas import tpu as pltpu

N_DEV = 4
P = 8

if os.environ.get("KERNEL_SCOPES"):
    _scope = jax.named_scope
else:
    def _scope(name):
        return contextlib.nullcontext()


def kernel(x, w_mat):
    m, k_per = x.shape
    _, n = w_mat.shape
    m_out = m // N_DEV
    nh = n // 2
    rp = m_out // P

    def body(x_ref, w_ref, out_ref, comm_r, comm_l,
             send_sems_r, recv_sems_r, send_sems_l, recv_sems_l):
        my = lax.axis_index("i")
        left = lax.rem(my + N_DEV - 1, N_DEV)
        right = lax.rem(my + 1, N_DEV)

        with _scope("barrier"):
            barrier_sem = pltpu.get_barrier_semaphore()
            for nbr in (left, right):
                pl.semaphore_signal(
                    barrier_sem, inc=1,
                    device_id=(nbr,), device_id_type=pl.DeviceIdType.MESH,
                )
            pl.semaphore_wait(barrier_sem, 2)

        def partial(c, col0, w_cols):
            xs = x_ref[pl.ds(c * m_out, m_out), :]
            return jnp.dot(xs, w_ref[:, col0:col0 + w_cols],
                           preferred_element_type=jnp.float32)

        def mk(comm, ssems, rsems, h, j, nbr):
            return pltpu.make_async_remote_copy(
                src_ref=comm.at[h, pl.ds(j * rp, rp), :],
                dst_ref=comm.at[h + 1, pl.ds(j * rp, rp), :],
                send_sem=ssems.at[h, j],
                recv_sem=rsems.at[h, j],
                device_id=(nbr,),
                device_id_type=pl.DeviceIdType.MESH,
            )

        rd = {}
        for h in range(N_DEV - 1):
            for j in range(P):
                rd["r", h, j] = mk(comm_r, send_sems_r, recv_sems_r, h, j, right)
                rd["l", h, j] = mk(comm_l, send_sems_l, recv_sems_l, h, j, left)

        with _scope("init_gemm"):
            comm_r[0, :, :] = partial(left, 0, nh)
            comm_l[0, :, :] = partial(right, nh, nh)

        with _scope("hop0_start"):
            for j in range(P):
                rd["r", 0, j].start()
                rd["l", 0, j].start()

        with _scope("p0_gemm"):
            p0 = partial(lax.rem(my + 2, N_DEV), 0, n)
        with _scope("hop0_recv_acc"):
            for j in range(P):
                r0, r1 = j * rp, (j + 1) * rp
                rd["r", 0, j].wait_recv()
                comm_r[1, r0:r1, :] = comm_r[1, r0:r1, :] + p0[r0:r1, 0:nh]
                rd["r", 1, j].start()
                rd["l", 0, j].wait_recv()
                comm_l[1, r0:r1, :] = comm_l[1, r0:r1, :] + p0[r0:r1, nh:n]
                rd["l", 1, j].start()

        with _scope("p1_gemm"):
            p1r = partial(lax.rem(my + 1, N_DEV), 0, nh)
            p1l = partial(lax.rem(my + 3, N_DEV), nh, nh)
        with _scope("hop1_recv_acc"):
            for j in range(P):
                r0, r1 = j * rp, (j + 1) * rp
                rd["r", 1, j].wait_recv()
                comm_r[2, r0:r1, :] = comm_r[2, r0:r1, :] + p1r[r0:r1, :]
                rd["r", 2, j].start()
                rd["l", 1, j].wait_recv()
                comm_l[2, r0:r1, :] = comm_l[2, r0:r1, :] + p1l[r0:r1, :]
                rd["l", 2, j].start()

        with _scope("p2_gemm"):
            p2 = partial(my, 0, n)
        with _scope("hop2_recv_store"):
            for j in range(P):
                r0, r1 = j * rp, (j + 1) * rp
                rd["r", 2, j].wait_recv()
                out_ref[r0:r1, 0:nh] = jnp.maximum(
                    comm_r[3, r0:r1, :] + p2[r0:r1, 0:nh], 0.0)
                rd["l", 2, j].wait_recv()
                out_ref[r0:r1, nh:n] = jnp.maximum(
                    comm_l[3, r0:r1, :] + p2[r0:r1, nh:n], 0.0)

        with _scope("drain_sends"):
            for h in range(N_DEV - 1):
                for j in range(P):
                    rd["r", h, j].wait_send()
                    rd["l", h, j].wait_send()

    return pl.pallas_call(
        body,
        out_shape=jax.ShapeDtypeStruct((m_out, n), jnp.float32),
        in_specs=[
            pl.BlockSpec(memory_space=pltpu.VMEM),
            pl.BlockSpec(memory_space=pltpu.VMEM),
        ],
        out_specs=pl.BlockSpec(memory_space=pltpu.VMEM),
        scratch_shapes=[
            pltpu.VMEM((N_DEV, m_out, nh), jnp.float32),
            pltpu.VMEM((N_DEV, m_out, nh), jnp.float32),
            pltpu.SemaphoreType.DMA((N_DEV - 1, P)),
            pltpu.SemaphoreType.DMA((N_DEV - 1, P)),
            pltpu.SemaphoreType.DMA((N_DEV - 1, P)),
            pltpu.SemaphoreType.DMA((N_DEV - 1, P)),
        ],
        compiler_params=pltpu.CompilerParams(collective_id=0),
    )(x, w_mat)


# device time: 80419 ns/iter; 1.0072x vs baseline; 1.0072x over previous
import jax
import jax.numpy as jnp
from jax import lax
from jax.experimental import pallas as pl
from jax.experimental.pallas import tpu as pltpu

N_DEV = 4
P = 4


def kernel(x, w_mat):
    m, k_per = x.shape
    _, n = w_mat.shape
    m_out = m // N_DEV
    nh = n // 2
    rp = m_out // P

    def body(x_ref, w_ref, out_ref,
             sA_r, sA_l, sBi_r, sBi_l, sBd_r, sBd_l,
             rA_l, rA_r, rBi_l, rBi_r, rBd_l, rBd_r,
             s_sems, r_sems, dep_s_r, dep_s_l, dep_r_l, dep_r_r):
        my = lax.axis_index("i")
        left = lax.rem(my + N_DEV - 1, N_DEV)
        right = lax.rem(my + 1, N_DEV)

        barrier_sem = pltpu.get_barrier_semaphore()
        for nbr in (left, right):
            pl.semaphore_signal(
                barrier_sem, inc=1,
                device_id=(nbr,), device_id_type=pl.DeviceIdType.MESH,
            )
        pl.semaphore_wait(barrier_sem, 2)

        def partial(c):
            xs = x_ref[pl.ds(c * m_out, m_out), :]
            return jnp.dot(xs, w_ref[:, :], preferred_element_type=jnp.float32)

        def copy(src, dst, ssem, rsem, dev):
            return pltpu.make_async_remote_copy(
                src_ref=src, dst_ref=dst, send_sem=ssem, recv_sem=rsem,
                device_id=(dev,), device_id_type=pl.DeviceIdType.MESH,
            )

        snd_A_r = copy(sA_r, rA_l, s_sems.at[0], r_sems.at[0], right)
        snd_A_l = copy(sA_l, rA_r, s_sems.at[1], r_sems.at[1], left)
        snd_Bi_r = copy(sBi_r, rBi_l, s_sems.at[2], r_sems.at[2], right)
        snd_Bi_l = copy(sBi_l, rBi_r, s_sems.at[3], r_sems.at[3], left)
        snd_Bd_r = [
            copy(sBd_r.at[pl.ds(j * rp, rp), :], rBd_l.at[pl.ds(j * rp, rp), :],
                 dep_s_r.at[j], dep_r_l.at[j], right)
            for j in range(P)
        ]
        snd_Bd_l = [
            copy(sBd_l.at[pl.ds(j * rp, rp), :], rBd_r.at[pl.ds(j * rp, rp), :],
                 dep_s_l.at[j], dep_r_r.at[j], left)
            for j in range(P)
        ]
        rcv_A_l = copy(rA_l, rA_l, s_sems.at[0], r_sems.at[0], left)
        rcv_A_r = copy(rA_r, rA_r, s_sems.at[1], r_sems.at[1], right)
        rcv_Bi_l = copy(rBi_l, rBi_l, s_sems.at[2], r_sems.at[2], left)
        rcv_Bi_r = copy(rBi_r, rBi_r, s_sems.at[3], r_sems.at[3], right)
        rcv_Bd_l = [
            copy(rBd_l.at[pl.ds(j * rp, rp), :], rBd_l.at[pl.ds(j * rp, rp), :],
                 dep_s_r.at[j], dep_r_l.at[j], left)
            for j in range(P)
        ]
        rcv_Bd_r = [
            copy(rBd_r.at[pl.ds(j * rp, rp), :], rBd_r.at[pl.ds(j * rp, rp), :],
                 dep_s_l.at[j], dep_r_r.at[j], right)
            for j in range(P)
        ]

        g2 = partial(lax.rem(my + 2, N_DEV))
        sA_r[:, :] = g2[:, 0:nh]
        sA_l[:, :] = g2[:, nh:n]
        snd_A_r.start()
        snd_A_l.start()

        g1 = partial(right)
        sBi_r[:, :] = g1[:, nh:n]
        snd_Bi_r.start()
        gm1 = partial(left)
        sBi_l[:, :] = gm1[:, 0:nh]
        snd_Bi_l.start()

        rcv_A_l.wait_recv()
        sBd_r[:, :] = g1[:, 0:nh] + rA_l[:, :]
        for j in range(P):
            snd_Bd_r[j].start()
        rcv_A_r.wait_recv()
        sBd_l[:, :] = gm1[:, nh:n] + rA_r[:, :]
        for j in range(P):
            snd_Bd_l[j].start()

        g0 = partial(my)

        rcv_Bi_r.wait_recv()
        rcv_Bi_l.wait_recv()
        for j in range(P):
            r0, r1 = j * rp, (j + 1) * rp
            rcv_Bd_l[j].wait_recv()
            out_ref[r0:r1, 0:nh] = jnp.maximum(
                g0[r0:r1, 0:nh] + rBi_r[r0:r1, :] + rBd_l[r0:r1, :], 0.0)
            rcv_Bd_r[j].wait_recv()
            out_ref[r0:r1, nh:n] = jnp.maximum(
                g0[r0:r1, nh:n] + rBi_l[r0:r1, :] + rBd_r[r0:r1, :], 0.0)

        for d in (snd_A_r, snd_A_l, snd_Bi_r, snd_Bi_l, *snd_Bd_r, *snd_Bd_l):
            d.wait_send()

    half = lambda: pltpu.VMEM((m_out, nh), jnp.float32)
    return pl.pallas_call(
        body,
        out_shape=jax.ShapeDtypeStruct((m_out, n), jnp.float32),
        in_specs=[
            pl.BlockSpec(memory_space=pltpu.VMEM),
            pl.BlockSpec(memory_space=pltpu.VMEM),
        ],
        out_specs=pl.BlockSpec(memory_space=pltpu.VMEM),
        scratch_shapes=[
            half(), half(), half(), half(), half(), half(),
            half(), half(), half(), half(), half(), half(),
            pltpu.SemaphoreType.DMA((4,)),
            pltpu.SemaphoreType.DMA((4,)),
            pltpu.SemaphoreType.DMA((P,)),
            pltpu.SemaphoreType.DMA((P,)),
            pltpu.SemaphoreType.DMA((P,)),
            pltpu.SemaphoreType.DMA((P,)),
        ],
        compiler_params=pltpu.CompilerParams(collective_id=0),
    )(x, w_mat)


# device time: 79599 ns/iter; 1.0176x vs baseline; 1.0103x over previous
import jax
import jax.numpy as jnp
from jax import lax
from jax.experimental import pallas as pl
from jax.experimental.pallas import tpu as pltpu

N_DEV = 4
P = 4


def kernel(x, w_mat):
    m, k_per = x.shape
    _, n = w_mat.shape
    m_out = m // N_DEV
    nh = n // 2
    rp = m_out // P

    def body(x_ref, w_ref, out_ref,
             sA_r, sA_l, sBi_r, sBi_l, sBd_r, sBd_l,
             rA_l, rA_r, rBi_l, rBi_r, rBd_l, rBd_r,
             s_sems, r_sems, a_s_r, a_s_l, a_r_l, a_r_r,
             dep_s_r, dep_s_l, dep_r_l, dep_r_r):
        my = lax.axis_index("i")
        left = lax.rem(my + N_DEV - 1, N_DEV)
        right = lax.rem(my + 1, N_DEV)

        barrier_sem = pltpu.get_barrier_semaphore()
        for nbr in (left, right):
            pl.semaphore_signal(
                barrier_sem, inc=1,
                device_id=(nbr,), device_id_type=pl.DeviceIdType.MESH,
            )
        pl.semaphore_wait(barrier_sem, 2)

        def partial(c, j=None):
            if j is None:
                xs = x_ref[pl.ds(c * m_out, m_out), :]
            else:
                xs = x_ref[pl.ds(c * m_out + j * rp, rp), :]
            return jnp.dot(xs, w_ref[:, :], preferred_element_type=jnp.float32)

        def copy(src, dst, ssem, rsem, dev):
            return pltpu.make_async_remote_copy(
                src_ref=src, dst_ref=dst, send_sem=ssem, recv_sem=rsem,
                device_id=(dev,), device_id_type=pl.DeviceIdType.MESH,
            )

        snd_A_r = [
            copy(sA_r.at[pl.ds(j * rp, rp), :], rA_l.at[pl.ds(j * rp, rp), :],
                 a_s_r.at[j], a_r_l.at[j], right)
            for j in range(P)
        ]
        snd_A_l = [
            copy(sA_l.at[pl.ds(j * rp, rp), :], rA_r.at[pl.ds(j * rp, rp), :],
                 a_s_l.at[j], a_r_r.at[j], left)
            for j in range(P)
        ]
        snd_Bi_r = copy(sBi_r, rBi_l, s_sems.at[0], r_sems.at[0], right)
        snd_Bi_l = copy(sBi_l, rBi_r, s_sems.at[1], r_sems.at[1], left)
        snd_Bd_r = [
            copy(sBd_r.at[pl.ds(j * rp, rp), :], rBd_l.at[pl.ds(j * rp, rp), :],
                 dep_s_r.at[j], dep_r_l.at[j], right)
            for j in range(P)
        ]
        snd_Bd_l = [
            copy(sBd_l.at[pl.ds(j * rp, rp), :], rBd_r.at[pl.ds(j * rp, rp), :],
                 dep_s_l.at[j], dep_r_r.at[j], left)
            for j in range(P)
        ]
        rcv_A_l = [
            copy(rA_l.at[pl.ds(j * rp, rp), :], rA_l.at[pl.ds(j * rp, rp), :],
                 a_s_r.at[j], a_r_l.at[j], left)
            for j in range(P)
        ]
        rcv_A_r = [
            copy(rA_r.at[pl.ds(j * rp, rp), :], rA_r.at[pl.ds(j * rp, rp), :],
                 a_s_l.at[j], a_r_r.at[j], right)
            for j in range(P)
        ]
        rcv_Bi_l = copy(rBi_l, rBi_l, s_sems.at[0], r_sems.at[0], left)
        rcv_Bi_r = copy(rBi_r, rBi_r, s_sems.at[1], r_sems.at[1], right)
        rcv_Bd_l = [
            copy(rBd_l.at[pl.ds(j * rp, rp), :], rBd_l.at[pl.ds(j * rp, rp), :],
                 dep_s_r.at[j], dep_r_l.at[j], left)
            for j in range(P)
        ]
        rcv_Bd_r = [
            copy(rBd_r.at[pl.ds(j * rp, rp), :], rBd_r.at[pl.ds(j * rp, rp), :],
                 dep_s_l.at[j], dep_r_r.at[j], right)
            for j in range(P)
        ]

        diag = lax.rem(my + 2, N_DEV)
        for j in range(P):
            r0, r1 = j * rp, (j + 1) * rp
            g2j = partial(diag, j)
            sA_r[r0:r1, :] = g2j[:, 0:nh]
            sA_l[r0:r1, :] = g2j[:, nh:n]
            snd_A_r[j].start()
            snd_A_l[j].start()

        g1 = partial(right)
        sBi_r[:, :] = g1[:, nh:n]
        snd_Bi_r.start()
        gm1 = partial(left)
        sBi_l[:, :] = gm1[:, 0:nh]
        snd_Bi_l.start()

        for j in range(P):
            r0, r1 = j * rp, (j + 1) * rp
            rcv_A_l[j].wait_recv()
            sBd_r[r0:r1, :] = g1[r0:r1, 0:nh] + rA_l[r0:r1, :]
            snd_Bd_r[j].start()
            rcv_A_r[j].wait_recv()
            sBd_l[r0:r1, :] = gm1[r0:r1, nh:n] + rA_r[r0:r1, :]
            snd_Bd_l[j].start()

        g0 = partial(my)

        rcv_Bi_r.wait_recv()
        rcv_Bi_l.wait_recv()
        for j in range(P):
            r0, r1 = j * rp, (j + 1) * rp
            rcv_Bd_l[j].wait_recv()
            out_ref[r0:r1, 0:nh] = jnp.maximum(
                g0[r0:r1, 0:nh] + rBi_r[r0:r1, :] + rBd_l[r0:r1, :], 0.0)
            rcv_Bd_r[j].wait_recv()
            out_ref[r0:r1, nh:n] = jnp.maximum(
                g0[r0:r1, nh:n] + rBi_l[r0:r1, :] + rBd_r[r0:r1, :], 0.0)

        for d in (*snd_A_r, *snd_A_l, snd_Bi_r, snd_Bi_l,
                  *snd_Bd_r, *snd_Bd_l):
            d.wait_send()

    half = lambda: pltpu.VMEM((m_out, nh), jnp.float32)
    return pl.pallas_call(
        body,
        out_shape=jax.ShapeDtypeStruct((m_out, n), jnp.float32),
        in_specs=[
            pl.BlockSpec(memory_space=pltpu.VMEM),
            pl.BlockSpec(memory_space=pltpu.VMEM),
        ],
        out_specs=pl.BlockSpec(memory_space=pltpu.VMEM),
        scratch_shapes=[
            half(), half(), half(), half(), half(), half(),
            half(), half(), half(), half(), half(), half(),
            pltpu.SemaphoreType.DMA((2,)),
            pltpu.SemaphoreType.DMA((2,)),
            pltpu.SemaphoreType.DMA((P,)),
            pltpu.SemaphoreType.DMA((P,)),
            pltpu.SemaphoreType.DMA((P,)),
            pltpu.SemaphoreType.DMA((P,)),
            pltpu.SemaphoreType.DMA((P,)),
            pltpu.SemaphoreType.DMA((P,)),
            pltpu.SemaphoreType.DMA((P,)),
            pltpu.SemaphoreType.DMA((P,)),
        ],
        compiler_params=pltpu.CompilerParams(collective_id=0),
    )(x, w_mat)
